# baseline (speedup 1.0000x reference)
import functools

import jax
import jax.numpy as jnp
from jax import lax
from jax.experimental import pallas as pl
from jax.experimental.pallas import tpu as pltpu
from jax.experimental.pallas import tpu_sc as plsc

_NC = 2
_NS = 16
_NW = _NC * _NS

_BATCH = 8
_CHW = 128 * 128 * 128
_PER_W = _CHW // 2
_CHUNK = 32 * 1024
_NCHUNK = _PER_W // _CHUNK
_TOTAL = _BATCH * 2 * _CHW


def _copy_region(src_hbm, out_hbm, k, half_off, bufs, lsems, ssems):
    src_off = k * _PER_W
    bb = k // 2
    hh = k % 2
    dst_off = bb * (2 * _CHW) + half_off + hh * _PER_W

    loads = [None] * _NCHUNK
    stores = [None] * _NCHUNK

    def load(i):
        return pltpu.async_copy(
            src_hbm.at[pl.ds(src_off + i * _CHUNK, _CHUNK)],
            bufs[i % 2], lsems[i % 2])

    def store(i):
        return pltpu.async_copy(
            bufs[i % 2],
            out_hbm.at[pl.ds(dst_off + i * _CHUNK, _CHUNK)],
            ssems[i % 2])

    loads[0] = load(0)
    for i in range(_NCHUNK):
        if i + 1 < _NCHUNK:
            if i >= 1:
                stores[i - 1].wait()
            loads[i + 1] = load(i + 1)
        loads[i].wait()
        stores[i] = store(i)
    stores[_NCHUNK - 2].wait()
    stores[_NCHUNK - 1].wait()


def _sc_body(a_hbm, b_hbm, out_hbm, buf0, buf1, ls0, ls1, ss0, ss1):
    wid = lax.axis_index("s") * _NC + lax.axis_index("c")
    bufs = (buf0, buf1)
    lsems = (ls0, ls1)
    ssems = (ss0, ss1)

    @pl.when(wid < _NS)
    def _():
        _copy_region(a_hbm, out_hbm, wid, 0, bufs, lsems, ssems)

    @pl.when(wid >= _NS)
    def _():
        _copy_region(b_hbm, out_hbm, wid - _NS, _CHW, bufs, lsems, ssems)


_sc_concat = functools.partial(
    pl.kernel,
    mesh=plsc.VectorSubcoreMesh(core_axis_name="c", subcore_axis_name="s"),
    out_type=jax.ShapeDtypeStruct((_TOTAL,), jnp.float32),
    scratch_types=[
        pltpu.VMEM((_CHUNK,), jnp.float32),
        pltpu.VMEM((_CHUNK,), jnp.float32),
        pltpu.SemaphoreType.DMA,
        pltpu.SemaphoreType.DMA,
        pltpu.SemaphoreType.DMA,
        pltpu.SemaphoreType.DMA,
    ],
)(_sc_body)


def kernel(a, b, scatter_a, scatter_b):
    B, C, H, W = a.shape
    out_flat = _sc_concat(a.reshape(-1), b.reshape(-1))
    return out_flat.reshape(B, 2 * C, H, W)
